# baseline (speedup 1.0000x reference)
import functools

import jax
import jax.numpy as jnp
from jax import lax
from jax.experimental import pallas as pl
from jax.experimental.pallas import tpu as pltpu
from jax.experimental.pallas import tpu_sc as plsc

D_, H_, W_ = 10, 200, 176
NVOX = D_ * H_ * W_
FINAL = 128
NEG = -1e30


Kt = 128
Tp = 40
T_REAL = 35


def _valid_rows(kt):
    t = lax.broadcasted_iota(jnp.int32, (kt, Tp, 1), 1)
    return (t < T_REAL).astype(jnp.float32)


def _layer1(xb, W1, b1):
    kt = xb.shape[0]
    x2 = xb.reshape(kt * Tp, 7)
    h = jnp.maximum(jnp.dot(x2, W1, preferred_element_type=jnp.float32) + b1, 0.0)
    return h


def _stats_pass1(x_ref, W1_ref, b1_ref, out_ref):
    i = pl.program_id(0)
    xb = x_ref[...]
    h = _layer1(xb, W1_ref[...], b1_ref[...])
    v = _valid_rows(xb.shape[0]).reshape(-1, 1)
    hv = h * v
    s = jnp.sum(hv, axis=0, keepdims=True)
    sq = jnp.sum(hv * hv, axis=0, keepdims=True)
    part = jnp.concatenate([s, sq], axis=0)

    @pl.when(i == 0)
    def _():
        out_ref[...] = jnp.zeros_like(out_ref)

    out_ref[...] += part


def _bn_coeffs(sums, g, bt, cnt):
    mean = sums[0:1, :] / cnt
    var = sums[1:2, :] / cnt - mean * mean
    s = g * lax.rsqrt(var + 1e-5)
    t = bt - mean * s
    return s, t


def _vfe_block(xb, h, sums, g, bt, cnt):
    kt = xb.shape[0]
    c = h.shape[-1]
    s, t = _bn_coeffs(sums, g, bt, cnt)
    hn = h * s + t
    h3 = hn.reshape(kt, Tp, c)
    vmask = _valid_rows(kt)
    hm = jnp.where(vmask > 0, h3, NEG)
    mp = jnp.max(hm, axis=1, keepdims=True)
    mp3 = jnp.broadcast_to(mp, h3.shape)
    cc = jnp.concatenate([h3, mp3], axis=2)
    pmask = (jnp.max(xb, axis=2, keepdims=True) != 0).astype(jnp.float32)
    return cc * pmask


def _stats_pass2(x_ref, W1_ref, b1_ref, g1_ref, bt1_ref, s1_ref,
                 W2_ref, b2_ref, out_ref, *, cnt):
    i = pl.program_id(0)
    xb = x_ref[...]
    kt = xb.shape[0]
    h1 = _layer1(xb, W1_ref[...], b1_ref[...])
    v1 = _vfe_block(xb, h1, s1_ref[...], g1_ref[...], bt1_ref[...], cnt)
    v1f = v1.reshape(kt * Tp, 32)
    h2 = jnp.maximum(
        jnp.dot(v1f, W2_ref[...], preferred_element_type=jnp.float32) + b2_ref[...],
        0.0)
    v = _valid_rows(kt).reshape(-1, 1)
    hv = h2 * v
    s = jnp.sum(hv, axis=0, keepdims=True)
    sq = jnp.sum(hv * hv, axis=0, keepdims=True)
    part = jnp.concatenate([s, sq], axis=0)

    @pl.when(i == 0)
    def _():
        out_ref[...] = jnp.zeros_like(out_ref)

    out_ref[...] += part


def _final_pass(x_ref, W1_ref, b1_ref, g1_ref, bt1_ref, s1_ref,
                W2_ref, b2_ref, g2_ref, bt2_ref, s2_ref,
                Wf_ref, bf_ref, o_ref, *, cnt):
    xb = x_ref[...]
    kt = xb.shape[0]
    h1 = _layer1(xb, W1_ref[...], b1_ref[...])
    v1 = _vfe_block(xb, h1, s1_ref[...], g1_ref[...], bt1_ref[...], cnt)
    v1f = v1.reshape(kt * Tp, 32)
    h2 = jnp.maximum(
        jnp.dot(v1f, W2_ref[...], preferred_element_type=jnp.float32) + b2_ref[...],
        0.0)
    v2 = _vfe_block(xb, h2, s2_ref[...], g2_ref[...], bt2_ref[...], cnt)
    v2f = v2.reshape(kt * Tp, FINAL)
    y = jnp.dot(v2f, Wf_ref[...], preferred_element_type=jnp.float32) + bf_ref[...]
    y3 = y.reshape(kt, Tp, FINAL)
    vmask = _valid_rows(kt)
    ym = jnp.where(vmask > 0, y3, NEG)
    o_ref[...] = jnp.max(ym, axis=1)


def _transpose_kernel(in_ref, out_ref):
    out_ref[...] = in_ref[...].T


def _vid_kernel(c_ref, out_ref):
    c = c_ref[...]
    out_ref[...] = c[:, 0:1] * (H_ * W_) + c[:, 1:2] * W_ + c[:, 2:3]


NW = 32
VPW = NVOX // NW
VPW_PAD = 11008
NPTS = 16384
NGRP = NPTS // 16
SENT = NPTS
GCH = 440
NCH = VPW // GCH


def _sc_build_idx(vid_hbm, idx_hbm, vid_v, table_v, keybuf_v, sem):
    wid = lax.axis_index("s") * 2 + lax.axis_index("c")
    base = wid * VPW
    pltpu.sync_copy(vid_hbm, vid_v)

    def init_body(j, _):
        table_v[pl.ds(j * 16, 16)] = jnp.full((16,), SENT, jnp.int32)
        return 0

    lax.fori_loop(0, VPW_PAD // 16, init_body, 0)
    keybuf_v[pl.ds(16, 16)] = jnp.full((16,), -1, jnp.int32)

    lane = lax.iota(jnp.int32, 16)

    def body(g, _):
        pi = g * 16 + lane
        vid = vid_v[pl.ds(g * 16, 16)]
        key = vid * 16 + lane
        sk, sv = plsc.sort_key_val(key, pi)
        keybuf_v[pl.ds(0, 16)] = sk
        nxt = keybuf_v[pl.ds(1, 16)]
        svid = lax.shift_right_arithmetic(sk, 4)
        nvid = lax.shift_right_arithmetic(nxt, 4)
        loc = svid - base
        m = (svid != nvid) & (loc >= 0) & (loc < VPW)
        locc = jnp.where(m, loc, 0)
        plsc.store_scatter(table_v, [locc], sv, mask=m)
        return 0

    lax.fori_loop(0, NGRP, body, 0)
    pltpu.sync_copy(table_v.at[pl.ds(0, VPW)], idx_hbm.at[pl.ds(base, VPW)])


def _sc_gather(opad_hbm, idx_hbm, out_hbm, idx_v, rows_v, sem):
    wid = lax.axis_index("s") * 2 + lax.axis_index("c")
    base = wid * VPW
    pltpu.sync_copy(idx_hbm.at[pl.ds(base, VPW)], idx_v)

    def body(j, _):
        off = pl.multiple_of(j * GCH, 8)
        pltpu.async_copy(opad_hbm.at[idx_v.at[pl.ds(off, GCH)]], rows_v, sem).wait()
        pltpu.sync_copy(rows_v, out_hbm.at[pl.ds(base + off, GCH)])
        return 0

    lax.fori_loop(0, NCH, body, 0)


def kernel(input, voxel_coor_buffer, W1, b1, g1, bt1, W2, b2, g2, bt2, Wf, bf):
    B, K, T, C = input.shape
    N = B * K
    cnt = float(N * T)

    x = input.reshape(N, T, C)
    xp = jnp.pad(x, ((0, 0), (0, Tp - T), (0, 0)))
    coor = voxel_coor_buffer.reshape(N, 3).astype(jnp.int32)

    b1r = b1.reshape(1, 16)
    g1r = g1.reshape(1, 16)
    bt1r = bt1.reshape(1, 16)
    b2r = b2.reshape(1, 64)
    g2r = g2.reshape(1, 64)
    bt2r = bt2.reshape(1, 64)
    bfr = bf.reshape(1, FINAL)

    grid = N // Kt
    full = lambda shp: pl.BlockSpec(shp, lambda i: (0,) * len(shp))
    xspec = pl.BlockSpec((Kt, Tp, C), lambda i: (i, 0, 0))

    sums1 = pl.pallas_call(
        _stats_pass1,
        grid=(grid,),
        in_specs=[xspec, full((7, 16)), full((1, 16))],
        out_specs=full((2, 16)),
        out_shape=jax.ShapeDtypeStruct((2, 16), jnp.float32),
    )(xp, W1, b1r)

    sums2 = pl.pallas_call(
        functools.partial(_stats_pass2, cnt=cnt),
        grid=(grid,),
        in_specs=[xspec, full((7, 16)), full((1, 16)), full((1, 16)),
                  full((1, 16)), full((2, 16)), full((32, 64)), full((1, 64))],
        out_specs=full((2, 64)),
        out_shape=jax.ShapeDtypeStruct((2, 64), jnp.float32),
    )(xp, W1, b1r, g1r, bt1r, sums1, W2, b2r)

    o = pl.pallas_call(
        functools.partial(_final_pass, cnt=cnt),
        grid=(grid,),
        in_specs=[xspec, full((7, 16)), full((1, 16)), full((1, 16)),
                  full((1, 16)), full((2, 16)), full((32, 64)), full((1, 64)),
                  full((1, 64)), full((1, 64)), full((2, 64)),
                  full((128, FINAL)), full((1, FINAL))],
        out_specs=pl.BlockSpec((Kt, FINAL), lambda i: (i, 0)),
        out_shape=jax.ShapeDtypeStruct((N, FINAL), jnp.float32),
    )(xp, W1, b1r, g1r, bt1r, sums1, W2, b2r, g2r, bt2r, sums2, Wf, bfr)

    o_pad = jnp.concatenate([o, jnp.zeros((8, FINAL), jnp.float32)], axis=0)

    vid = pl.pallas_call(
        _vid_kernel,
        in_specs=[pl.BlockSpec((N, 3), lambda: (0, 0))],
        out_specs=pl.BlockSpec((N, 1), lambda: (0, 0)),
        out_shape=jax.ShapeDtypeStruct((N, 1), jnp.int32),
    )(coor).reshape(N)

    mesh = plsc.VectorSubcoreMesh(core_axis_name="c", subcore_axis_name="s")

    idx = pl.kernel(
        _sc_build_idx,
        mesh=mesh,
        compiler_params=pltpu.CompilerParams(needs_layout_passes=False),
        out_type=jax.ShapeDtypeStruct((NVOX,), jnp.int32),
        scratch_types=[
            pltpu.VMEM((N,), jnp.int32),
            pltpu.VMEM((VPW_PAD,), jnp.int32),
            pltpu.VMEM((32,), jnp.int32),
            pltpu.SemaphoreType.DMA,
        ],
    )(vid)

    dense = pl.kernel(
        _sc_gather,
        mesh=mesh,
        compiler_params=pltpu.CompilerParams(needs_layout_passes=False),
        out_type=jax.ShapeDtypeStruct((NVOX, FINAL), jnp.float32),
        scratch_types=[
            pltpu.VMEM((VPW,), jnp.int32),
            pltpu.VMEM((GCH, FINAL), jnp.float32),
            pltpu.SemaphoreType.DMA,
        ],
    )(o_pad, idx)

    TT = 3200
    outT = pl.pallas_call(
        _transpose_kernel,
        grid=(NVOX // TT,),
        in_specs=[pl.BlockSpec((TT, FINAL), lambda i: (i, 0))],
        out_specs=pl.BlockSpec((FINAL, TT), lambda i: (0, i)),
        out_shape=jax.ShapeDtypeStruct((FINAL, NVOX), jnp.float32),
    )(dense)

    return outT.reshape(1, FINAL, D_, H_, W_)


# speedup vs baseline: 8.1217x; 8.1217x over previous
import functools

import jax
import jax.numpy as jnp
from jax import lax
from jax.experimental import pallas as pl
from jax.experimental.pallas import tpu as pltpu
from jax.experimental.pallas import tpu_sc as plsc

D_, H_, W_ = 10, 200, 176
NVOX = D_ * H_ * W_
FINAL = 128
NEG = -1e30


Kt = 128
Tp = 40
T_REAL = 35


def _valid_rows(kt):
    t = lax.broadcasted_iota(jnp.int32, (kt, Tp, 1), 1)
    return (t < T_REAL).astype(jnp.float32)


def _layer1(xb, W1, b1):
    kt = xb.shape[0]
    x2 = xb.reshape(kt * Tp, 7)
    h = jnp.maximum(jnp.dot(x2, W1, preferred_element_type=jnp.float32) + b1, 0.0)
    return h


def _stats_pass1(x_ref, W1_ref, b1_ref, out_ref):
    i = pl.program_id(0)
    xb = x_ref[...]
    h = _layer1(xb, W1_ref[...], b1_ref[...])
    v = _valid_rows(xb.shape[0]).reshape(-1, 1)
    hv = h * v
    s = jnp.sum(hv, axis=0, keepdims=True)
    sq = jnp.sum(hv * hv, axis=0, keepdims=True)
    part = jnp.concatenate([s, sq], axis=0)

    @pl.when(i == 0)
    def _():
        out_ref[...] = jnp.zeros_like(out_ref)

    out_ref[...] += part


def _bn_coeffs(sums, g, bt, cnt):
    mean = sums[0:1, :] / cnt
    var = sums[1:2, :] / cnt - mean * mean
    s = g * lax.rsqrt(var + 1e-5)
    t = bt - mean * s
    return s, t


def _vfe_block(xb, h, sums, g, bt, cnt):
    kt = xb.shape[0]
    c = h.shape[-1]
    s, t = _bn_coeffs(sums, g, bt, cnt)
    hn = h * s + t
    h3 = hn.reshape(kt, Tp, c)
    vmask = _valid_rows(kt)
    hm = jnp.where(vmask > 0, h3, NEG)
    mp = jnp.max(hm, axis=1, keepdims=True)
    mp3 = jnp.broadcast_to(mp, h3.shape)
    cc = jnp.concatenate([h3, mp3], axis=2)
    pmask = (jnp.max(xb, axis=2, keepdims=True) != 0).astype(jnp.float32)
    return cc * pmask


def _stats_pass2(x_ref, W1_ref, b1_ref, g1_ref, bt1_ref, s1_ref,
                 W2_ref, b2_ref, out_ref, *, cnt):
    i = pl.program_id(0)
    xb = x_ref[...]
    kt = xb.shape[0]
    h1 = _layer1(xb, W1_ref[...], b1_ref[...])
    v1 = _vfe_block(xb, h1, s1_ref[...], g1_ref[...], bt1_ref[...], cnt)
    v1f = v1.reshape(kt * Tp, 32)
    h2 = jnp.maximum(
        jnp.dot(v1f, W2_ref[...], preferred_element_type=jnp.float32) + b2_ref[...],
        0.0)
    v = _valid_rows(kt).reshape(-1, 1)
    hv = h2 * v
    s = jnp.sum(hv, axis=0, keepdims=True)
    sq = jnp.sum(hv * hv, axis=0, keepdims=True)
    part = jnp.concatenate([s, sq], axis=0)

    @pl.when(i == 0)
    def _():
        out_ref[...] = jnp.zeros_like(out_ref)

    out_ref[...] += part


def _final_pass(x_ref, W1_ref, b1_ref, g1_ref, bt1_ref, s1_ref,
                W2_ref, b2_ref, g2_ref, bt2_ref, s2_ref,
                Wf_ref, bf_ref, o_ref, *, cnt):
    xb = x_ref[...]
    kt = xb.shape[0]
    h1 = _layer1(xb, W1_ref[...], b1_ref[...])
    v1 = _vfe_block(xb, h1, s1_ref[...], g1_ref[...], bt1_ref[...], cnt)
    v1f = v1.reshape(kt * Tp, 32)
    h2 = jnp.maximum(
        jnp.dot(v1f, W2_ref[...], preferred_element_type=jnp.float32) + b2_ref[...],
        0.0)
    v2 = _vfe_block(xb, h2, s2_ref[...], g2_ref[...], bt2_ref[...], cnt)
    v2f = v2.reshape(kt * Tp, FINAL)
    y = jnp.dot(v2f, Wf_ref[...], preferred_element_type=jnp.float32) + bf_ref[...]
    y3 = y.reshape(kt, Tp, FINAL)
    vmask = _valid_rows(kt)
    ym = jnp.where(vmask > 0, y3, NEG)
    o_ref[...] = jnp.max(ym, axis=1)


def _transpose_kernel(in_ref, idx_ref, out_ref):
    valid = (idx_ref[0, 0, :] != SENT)[None, :]
    out_ref[...] = jnp.where(valid, in_ref[...].T, 0.0)


def _vid_kernel(c_ref, out_ref):
    c = c_ref[...]
    out_ref[...] = c[:, 0:1] * (H_ * W_) + c[:, 1:2] * W_ + c[:, 2:3]


NW = 32
VPW = NVOX // NW
VPW_PAD = 11008
NPTS = 16384
NGRP = NPTS // 16
SENT = NPTS
CCH = 256
NCHMAX = 44
DENSE_ROWS = NVOX + 512


def _sc_scatter(vid_hbm, opad_hbm, idx_hbm, dense_hbm,
                vid_v, table_v, keybuf_v, plist_v, vlist_v, rows_v,
                semg, sems):
    wid = lax.axis_index("s") * 2 + lax.axis_index("c")
    base = wid * VPW
    pltpu.sync_copy(vid_hbm, vid_v)

    def init_body(j, _):
        table_v[pl.ds(j * 16, 16)] = jnp.full((16,), SENT, jnp.int32)
        return 0

    lax.fori_loop(0, VPW_PAD // 16, init_body, 0)
    keybuf_v[pl.ds(16, 16)] = jnp.full((16,), -1, jnp.int32)

    lane = lax.iota(jnp.int32, 16)

    def body(g, _):
        pi = g * 16 + lane
        vid = vid_v[pl.ds(g * 16, 16)]
        key = vid * 16 + lane
        sk, sv = plsc.sort_key_val(key, pi)
        keybuf_v[pl.ds(0, 16)] = sk
        nxt = keybuf_v[pl.ds(1, 16)]
        svid = lax.shift_right_arithmetic(sk, 4)
        nvid = lax.shift_right_arithmetic(nxt, 4)
        loc = svid - base
        m = (svid != nvid) & (loc >= 0) & (loc < VPW)
        locc = jnp.where(m, loc, 0)
        plsc.store_scatter(table_v, [locc], sv, mask=m)
        return 0

    lax.fori_loop(0, NGRP, body, 0)
    pltpu.sync_copy(table_v.at[pl.ds(0, VPW)], idx_hbm.at[pl.ds(base, VPW)])

    def compact(g, cnt):
        t = table_v[pl.ds(g * 16, 16)]
        occ = t != SENT
        pos = cnt + plsc.cumsum(jnp.where(occ, 1, 0)) - 1
        posc = jnp.where(occ, pos, 0)
        plsc.store_scatter(plist_v, [posc], t, mask=occ)
        plsc.store_scatter(vlist_v, [posc], base + g * 16 + lane, mask=occ)
        return cnt + plsc.all_reduce_population_count(occ)

    cntv = lax.fori_loop(0, VPW_PAD // 16, compact,
                         jnp.zeros((16,), jnp.int32))

    for k in range(17):
        pos = cntv + k * 16 + lane
        plsc.store_scatter(plist_v, [pos], jnp.full((16,), SENT, jnp.int32))
        plsc.store_scatter(vlist_v, [pos],
                           jnp.full((16,), NVOX + k * 16, jnp.int32) + lane)

    cnt_s = jnp.max(cntv)
    nch = (cnt_s + (CCH - 1)) // CCH
    gpc = CCH // 16

    def chunk_body(c, _):
        co = c * CCH
        waits = []
        for g in range(gpc):
            pvec = plist_v[pl.ds(co + g * 16, 16)]
            waits.append(pltpu.async_copy(
                opad_hbm.at[pvec], rows_v.at[pl.ds(g * 16, 16)], semg))
        for w in waits:
            w.wait()
        waits = []
        for g in range(gpc):
            vvec = vlist_v[pl.ds(co + g * 16, 16)]
            waits.append(pltpu.async_copy(
                rows_v.at[pl.ds(g * 16, 16)], dense_hbm.at[vvec], sems))
        for w in waits:
            w.wait()
        return 0

    lax.fori_loop(0, nch, chunk_body, 0)


def kernel(input, voxel_coor_buffer, W1, b1, g1, bt1, W2, b2, g2, bt2, Wf, bf):
    B, K, T, C = input.shape
    N = B * K
    cnt = float(N * T)

    x = input.reshape(N, T, C)
    xp = jnp.pad(x, ((0, 0), (0, Tp - T), (0, 0)))
    coor = voxel_coor_buffer.reshape(N, 3).astype(jnp.int32)

    b1r = b1.reshape(1, 16)
    g1r = g1.reshape(1, 16)
    bt1r = bt1.reshape(1, 16)
    b2r = b2.reshape(1, 64)
    g2r = g2.reshape(1, 64)
    bt2r = bt2.reshape(1, 64)
    bfr = bf.reshape(1, FINAL)

    grid = N // Kt
    full = lambda shp: pl.BlockSpec(shp, lambda i: (0,) * len(shp))
    xspec = pl.BlockSpec((Kt, Tp, C), lambda i: (i, 0, 0))

    sums1 = pl.pallas_call(
        _stats_pass1,
        grid=(grid,),
        in_specs=[xspec, full((7, 16)), full((1, 16))],
        out_specs=full((2, 16)),
        out_shape=jax.ShapeDtypeStruct((2, 16), jnp.float32),
    )(xp, W1, b1r)

    sums2 = pl.pallas_call(
        functools.partial(_stats_pass2, cnt=cnt),
        grid=(grid,),
        in_specs=[xspec, full((7, 16)), full((1, 16)), full((1, 16)),
                  full((1, 16)), full((2, 16)), full((32, 64)), full((1, 64))],
        out_specs=full((2, 64)),
        out_shape=jax.ShapeDtypeStruct((2, 64), jnp.float32),
    )(xp, W1, b1r, g1r, bt1r, sums1, W2, b2r)

    o = pl.pallas_call(
        functools.partial(_final_pass, cnt=cnt),
        grid=(grid,),
        in_specs=[xspec, full((7, 16)), full((1, 16)), full((1, 16)),
                  full((1, 16)), full((2, 16)), full((32, 64)), full((1, 64)),
                  full((1, 64)), full((1, 64)), full((2, 64)),
                  full((128, FINAL)), full((1, FINAL))],
        out_specs=pl.BlockSpec((Kt, FINAL), lambda i: (i, 0)),
        out_shape=jax.ShapeDtypeStruct((N, FINAL), jnp.float32),
    )(xp, W1, b1r, g1r, bt1r, sums1, W2, b2r, g2r, bt2r, sums2, Wf, bfr)

    o_pad = jnp.concatenate([o, jnp.zeros((8, FINAL), jnp.float32)], axis=0)

    vid = pl.pallas_call(
        _vid_kernel,
        in_specs=[pl.BlockSpec((N, 3), lambda: (0, 0))],
        out_specs=pl.BlockSpec((N, 1), lambda: (0, 0)),
        out_shape=jax.ShapeDtypeStruct((N, 1), jnp.int32),
    )(coor).reshape(N)

    mesh = plsc.VectorSubcoreMesh(core_axis_name="c", subcore_axis_name="s")

    idx, dense = pl.kernel(
        _sc_scatter,
        mesh=mesh,
        compiler_params=pltpu.CompilerParams(needs_layout_passes=False),
        out_type=(
            jax.ShapeDtypeStruct((NVOX,), jnp.int32),
            jax.ShapeDtypeStruct((DENSE_ROWS, FINAL), jnp.float32),
        ),
        scratch_types=[
            pltpu.VMEM((N,), jnp.int32),
            pltpu.VMEM((VPW_PAD,), jnp.int32),
            pltpu.VMEM((32,), jnp.int32),
            pltpu.VMEM(((NCHMAX + 1) * CCH,), jnp.int32),
            pltpu.VMEM(((NCHMAX + 1) * CCH,), jnp.int32),
            pltpu.VMEM((CCH, FINAL), jnp.float32),
            pltpu.SemaphoreType.DMA,
            pltpu.SemaphoreType.DMA,
        ],
    )(vid, o_pad)

    TT = 3200
    idx3 = idx.reshape(NVOX // TT, 1, TT)
    outT = pl.pallas_call(
        _transpose_kernel,
        grid=(NVOX // TT,),
        in_specs=[pl.BlockSpec((TT, FINAL), lambda i: (i, 0)),
                  pl.BlockSpec((1, 1, TT), lambda i: (i, 0, 0))],
        out_specs=pl.BlockSpec((FINAL, TT), lambda i: (0, i)),
        out_shape=jax.ShapeDtypeStruct((FINAL, NVOX), jnp.float32),
    )(dense, idx3)

    return outT.reshape(1, FINAL, D_, H_, W_)
